# Initial kernel scaffold; baseline (speedup 1.0000x reference)
import functools

import jax
import jax.numpy as jnp
from jax import lax
from jax.experimental import pallas as pl
from jax.experimental.pallas import tpu as pltpu
from jax.experimental.pallas import tpu_sc as plsc

H, W, DIM = 480, 640, 32
B, N = 16, 65536
PIX = H * W
NC, NS, L = 2, 16, 16
NBPC = B // NC
CHUNK = N // NS
NROW = CHUNK // 128
GS = PIX // NS
ZCH = 2400
TAPS = [(dy, dx) for dy in (-1, 0, 1) for dx in (-1, 0, 1)]
BLK = 4096


def _sc_coef(xt):
    mesh = plsc.VectorSubcoreMesh(
        core_axis_name="c", subcore_axis_name="s",
        num_cores=NC, num_subcores=NS)

    @functools.partial(
        pl.kernel,
        out_type=jax.ShapeDtypeStruct((B, N, 16), jnp.float32),
        mesh=mesh,
        scratch_types=[
            pltpu.VMEM((CHUNK,), jnp.float32),
            pltpu.VMEM((CHUNK,), jnp.float32),
            pltpu.VMEM((CHUNK,), jnp.float32),
            pltpu.VMEM((CHUNK,), jnp.float32),
            pltpu.VMEM((CHUNK,), jnp.int32),
            pltpu.VMEM((CHUNK,), jnp.int32),
            pltpu.VMEM((NROW, 128), jnp.int32),
            pltpu.VMEM((NROW, 128), jnp.int32),
            pltpu.VMEM((NROW, 128), jnp.float32),
            pltpu.VMEM((CHUNK, 16), jnp.float32),
            pltpu.VMEM((128,), jnp.float32),
            pltpu.VMEM((ZCH,), jnp.float32),
            pltpu.VMEM_SHARED((PIX,), jnp.float32),
            pltpu.SemaphoreType.DMA,
            pltpu.SemaphoreType.DMA,
        ],
    )
    def k(xt_hbm, out_hbm, x0_v, x1_v, x2_v, x3_v, px_v, py_v, idx_v,
          nidx_v, gath_v, coef_v, ones_v, zeros_v, grid_sh, sem_s, sem_g):
        c = lax.axis_index("c")
        s = lax.axis_index("s")
        base = s * CHUNK

        def fill_ones(i, carry):
            ones_v[pl.ds(i * L, L)] = jnp.full((L,), 1.0, jnp.float32)
            return carry
        lax.fori_loop(0, 128 // L, fill_ones, 0)

        def fill_zeros(i, carry):
            zeros_v[pl.ds(i * L, L)] = jnp.zeros((L,), jnp.float32)
            return carry
        lax.fori_loop(0, ZCH // L, fill_zeros, 0)

        def fill_const_cols(i, carry):
            e = i * L + lax.iota(jnp.int32, (L,))
            plsc.store_scatter(
                coef_v, [e, jnp.full((L,), 12, jnp.int32)],
                jnp.full((L,), 1.0, jnp.float32))
            for col in (13, 14, 15):
                plsc.store_scatter(
                    coef_v, [e, jnp.full((L,), col, jnp.int32)],
                    jnp.zeros((L,), jnp.float32))
            return carry
        lax.fori_loop(0, CHUNK // L, fill_const_cols, 0)

        def batch_body(kk, carry):
            b = c * NBPC + kk
            def zloop(j, cz):
                pltpu.sync_copy(zeros_v,
                                grid_sh.at[pl.ds(s * GS + j * ZCH, ZCH)])
                return cz
            lax.fori_loop(0, GS // ZCH, zloop, 0)

            pltpu.sync_copy(xt_hbm.at[0, b, pl.ds(base, CHUNK)], x0_v)
            pltpu.sync_copy(xt_hbm.at[1, b, pl.ds(base, CHUNK)], x1_v)

            def comp(i, cc):
                for k8 in range(8):
                    sl = pl.ds(i * 128 + k8 * L, L)
                    px = jnp.clip((x0_v[sl] * float(W - 1))
                                  .astype(jnp.int32), 0, W - 1)
                    py = jnp.clip((x1_v[sl] * float(H - 1))
                                  .astype(jnp.int32), 0, H - 1)
                    px_v[sl] = px
                    py_v[sl] = py
                    idx_v[i, pl.ds(k8 * L, L)] = py * W + px
                return cc
            lax.fori_loop(0, NROW, comp, 0)

            plsc.subcore_barrier()

            descs = [
                pltpu.async_copy(ones_v, grid_sh.at[idx_v.at[j]], sem_s,
                                 add=True)
                for j in range(NROW)
            ]
            for d in descs:
                d.wait()

            pltpu.sync_copy(xt_hbm.at[2, b, pl.ds(base, CHUNK)], x2_v)
            pltpu.sync_copy(xt_hbm.at[3, b, pl.ds(base, CHUNK)], x3_v)

            def tpcols(i, cc):
                sl = pl.ds(i * L, L)
                e = i * L + lax.iota(jnp.int32, (L,))
                t = x2_v[sl]
                p = x3_v[sl]
                plsc.store_scatter(
                    coef_v, [e, jnp.full((L,), 9, jnp.int32)], t)
                plsc.store_scatter(
                    coef_v, [e, jnp.full((L,), 10, jnp.int32)], p * t)
                plsc.store_scatter(
                    coef_v, [e, jnp.full((L,), 11, jnp.int32)], p)
                return cc
            lax.fori_loop(0, CHUNK // L, tpcols, 0)

            plsc.subcore_barrier()

            for tap, (dy, dx) in enumerate(TAPS):
                def nloop(i, cc, dy=dy, dx=dx):
                    for k8 in range(8):
                        sl = pl.ds(i * 128 + k8 * L, L)
                        px = px_v[sl]
                        py = py_v[sl]
                        if dy < 0:
                            ny = jnp.maximum(py - 1, 0)
                        elif dy > 0:
                            ny = jnp.minimum(py + 1, H - 1)
                        else:
                            ny = py
                        if dx < 0:
                            nx = jnp.maximum(px - 1, 0)
                        elif dx > 0:
                            nx = jnp.minimum(px + 1, W - 1)
                        else:
                            nx = px
                        nidx_v[i, pl.ds(k8 * L, L)] = ny * W + nx
                    return cc
                lax.fori_loop(0, NROW, nloop, 0)

                gds = [
                    pltpu.async_copy(grid_sh.at[nidx_v.at[j]],
                                     gath_v.at[j], sem_g)
                    for j in range(NROW)
                ]
                for d in gds:
                    d.wait()

                def gcol(i, tap=tap):
                    e = i * L + lax.iota(jnp.int32, (L,))
                    v = gath_v[i // 8, pl.ds((i % 8) * L, L)]
                    plsc.store_scatter(
                        coef_v, [e, jnp.full((L,), tap, jnp.int32)], v)

                def gcol_loop(i, cc, tap=tap):
                    e = i * L + lax.iota(jnp.int32, (L,))
                    v = gath_v[lax.div(i, 8), pl.ds(lax.rem(i, 8) * L, L)]
                    plsc.store_scatter(
                        coef_v, [e, jnp.full((L,), tap, jnp.int32)], v)
                    return cc
                lax.fori_loop(0, CHUNK // L, gcol_loop, 0)

            pltpu.sync_copy(coef_v, out_hbm.at[b, pl.ds(base, CHUNK)])
            plsc.subcore_barrier()
            return carry

        lax.fori_loop(0, NBPC, batch_body, 0)

    return k(xt)


def _tc_body(kern_ref, wproj_ref, wp_ref, wn_ref, bp_ref, bn_ref,
             coef_ref, out_ref):
    wproj = wproj_ref[...]
    A = wproj[:, :DIM]
    Bm = wproj[:, DIM:]
    kern = kern_ref[...]
    K2 = lax.dot_general(kern, A, (((1,), (1,)), ((), ())),
                         preferred_element_type=jnp.float32)
    up = jnp.sum(Bm * wp_ref[...], axis=1)
    un = jnp.sum(Bm * wn_ref[...], axis=1)
    cp = jnp.sum(Bm * bp_ref[...], axis=1)
    cn = jnp.sum(Bm * bn_ref[...], axis=1)
    M = jnp.concatenate(
        [K2, un[None, :], (up - un)[None, :], (cp - cn)[None, :],
         cn[None, :], jnp.zeros((3, DIM), jnp.float32)], axis=0)
    out_ref[0] = lax.dot_general(
        coef_ref[0], M, (((1,), (0,)), ((), ())),
        preferred_element_type=jnp.float32)


def _tc_combine(coef, kern, Wproj, wp_row, wn_row, bp_row, bn_row):
    def wspec(shape):
        return pl.BlockSpec(shape, lambda b, n: (0,) * len(shape))
    return pl.pallas_call(
        _tc_body,
        grid=(B, N // BLK),
        in_specs=[
            wspec((9, DIM)),
            wspec((DIM, 2 * DIM)),
            wspec((1, DIM)),
            wspec((1, DIM)),
            wspec((1, DIM)),
            wspec((1, DIM)),
            pl.BlockSpec((1, BLK, 16), lambda b, n: (b, n, 0)),
        ],
        out_specs=pl.BlockSpec((1, BLK, DIM), lambda b, n: (b, n, 0)),
        out_shape=jax.ShapeDtypeStruct((B, N, DIM), jnp.float32),
    )(kern, Wproj, wp_row, wn_row, bp_row, bn_row, coef)


def kernel(x, kernel, Wp, bp, Wn, bn, Wproj):
    xt = jnp.transpose(x, (2, 0, 1))
    coef = _sc_coef(xt)
    return _tc_combine(coef, kernel, Wproj,
                       Wp[:, 0].reshape(1, DIM), Wn[:, 0].reshape(1, DIM),
                       bp.reshape(1, DIM), bn.reshape(1, DIM))


# speedup vs baseline: 2.3196x; 2.3196x over previous
import functools

import jax
import jax.numpy as jnp
from jax import lax
from jax.experimental import pallas as pl
from jax.experimental.pallas import tpu as pltpu
from jax.experimental.pallas import tpu_sc as plsc

H, W, DIM = 480, 640, 32
B, N = 16, 65536
PIX = H * W
NC, NS, L = 2, 16, 16
NBPC = B // NC
CHUNK = N // NS
NROW = CHUNK // 128
GS = PIX // NS
ZCH = 2400
TAPS = [(dy, dx) for dy in (-1, 0, 1) for dx in (-1, 0, 1)]
BLK = 4096


def _sc_coef(xt):
    mesh = plsc.VectorSubcoreMesh(
        core_axis_name="c", subcore_axis_name="s",
        num_cores=NC, num_subcores=NS)

    @functools.partial(
        pl.kernel,
        out_type=jax.ShapeDtypeStruct((B, N * 16), jnp.float32),
        mesh=mesh,
        compiler_params=pltpu.CompilerParams(needs_layout_passes=False),
        scratch_types=[
            pltpu.VMEM((CHUNK,), jnp.float32),
            pltpu.VMEM((CHUNK,), jnp.float32),
            pltpu.VMEM((CHUNK,), jnp.float32),
            pltpu.VMEM((CHUNK,), jnp.float32),
            pltpu.VMEM((CHUNK,), jnp.int32),
            pltpu.VMEM((CHUNK,), jnp.int32),
            pltpu.VMEM((NROW, 128), jnp.int32),
            pltpu.VMEM((NROW, 128), jnp.int32),
            pltpu.VMEM((NROW, 128), jnp.float32),
            pltpu.VMEM((CHUNK * 16,), jnp.float32),
            pltpu.VMEM((128,), jnp.float32),
            pltpu.VMEM((ZCH,), jnp.float32),
            pltpu.VMEM_SHARED((PIX,), jnp.float32),
            pltpu.SemaphoreType.DMA,
            pltpu.SemaphoreType.DMA,
        ],
    )
    def k(xt_hbm, out_hbm, x0_v, x1_v, x2_v, x3_v, px_v, py_v, idx_v,
          nidx_v, gath_v, coef_v, ones_v, zeros_v, grid_sh, sem_s, sem_g):
        c = lax.axis_index("c")
        s = lax.axis_index("s")
        base = s * CHUNK

        def fill_ones(i, carry):
            ones_v[pl.ds(i * L, L)] = jnp.full((L,), 1.0, jnp.float32)
            return carry
        lax.fori_loop(0, 128 // L, fill_ones, 0)

        def fill_zeros(i, carry):
            zeros_v[pl.ds(i * L, L)] = jnp.zeros((L,), jnp.float32)
            return carry
        lax.fori_loop(0, ZCH // L, fill_zeros, 0)

        def fill_const_cols(i, carry):
            e16 = i * (L * 16) + lax.iota(jnp.int32, L) * 16
            plsc.store_scatter(coef_v, [e16 + 12],
                               jnp.full((L,), 1.0, jnp.float32))
            for col in (13, 14, 15):
                plsc.store_scatter(coef_v, [e16 + col],
                                   jnp.zeros((L,), jnp.float32))
            return carry
        lax.fori_loop(0, CHUNK // L, fill_const_cols, 0)

        def batch_body(kk, carry):
            b = c * NBPC + kk
            def zloop(j, cz):
                pltpu.sync_copy(zeros_v,
                                grid_sh.at[pl.ds(s * GS + j * ZCH, ZCH)])
                return cz
            lax.fori_loop(0, GS // ZCH, zloop, 0)

            pltpu.sync_copy(xt_hbm.at[0, b, pl.ds(base, CHUNK)], x0_v)
            pltpu.sync_copy(xt_hbm.at[1, b, pl.ds(base, CHUNK)], x1_v)

            def comp(i, cc):
                for k8 in range(8):
                    sl = pl.ds(i * 128 + k8 * L, L)
                    px = jnp.clip((x0_v[sl] * float(W - 1))
                                  .astype(jnp.int32), 0, W - 1)
                    py = jnp.clip((x1_v[sl] * float(H - 1))
                                  .astype(jnp.int32), 0, H - 1)
                    px_v[sl] = px
                    py_v[sl] = py
                    idx_v[i, pl.ds(k8 * L, L)] = py * W + px
                return cc
            lax.fori_loop(0, NROW, comp, 0)

            plsc.subcore_barrier()

            descs = [
                pltpu.async_copy(ones_v, grid_sh.at[idx_v.at[j]], sem_s,
                                 add=True)
                for j in range(NROW)
            ]
            for d in descs:
                d.wait()

            pltpu.sync_copy(xt_hbm.at[2, b, pl.ds(base, CHUNK)], x2_v)
            pltpu.sync_copy(xt_hbm.at[3, b, pl.ds(base, CHUNK)], x3_v)

            def tpcols(i, cc):
                sl = pl.ds(i * L, L)
                e16 = i * (L * 16) + lax.iota(jnp.int32, L) * 16
                t = x2_v[sl]
                p = x3_v[sl]
                plsc.store_scatter(coef_v, [e16 + 9], t)
                plsc.store_scatter(coef_v, [e16 + 10], p * t)
                plsc.store_scatter(coef_v, [e16 + 11], p)
                return cc
            lax.fori_loop(0, CHUNK // L, tpcols, 0)

            plsc.subcore_barrier()

            for tap, (dy, dx) in enumerate(TAPS):
                def nloop(i, cc, dy=dy, dx=dx):
                    for k8 in range(8):
                        sl = pl.ds(i * 128 + k8 * L, L)
                        px = px_v[sl]
                        py = py_v[sl]
                        if dy < 0:
                            ny = jnp.maximum(py - 1, 0)
                        elif dy > 0:
                            ny = jnp.minimum(py + 1, H - 1)
                        else:
                            ny = py
                        if dx < 0:
                            nx = jnp.maximum(px - 1, 0)
                        elif dx > 0:
                            nx = jnp.minimum(px + 1, W - 1)
                        else:
                            nx = px
                        nidx_v[i, pl.ds(k8 * L, L)] = ny * W + nx
                    return cc
                lax.fori_loop(0, NROW, nloop, 0)

                gds = [
                    pltpu.async_copy(grid_sh.at[nidx_v.at[j]],
                                     gath_v.at[j], sem_g)
                    for j in range(NROW)
                ]
                for d in gds:
                    d.wait()

                def gcol_loop(i, cc, tap=tap):
                    e16 = i * (L * 16) + lax.iota(jnp.int32, L) * 16
                    v = gath_v[lax.div(i, 8), pl.ds(lax.rem(i, 8) * L, L)]
                    plsc.store_scatter(coef_v, [e16 + tap], v)
                    return cc
                lax.fori_loop(0, CHUNK // L, gcol_loop, 0)

            pltpu.sync_copy(coef_v,
                            out_hbm.at[b, pl.ds(base * 16, CHUNK * 16)])
            plsc.subcore_barrier()
            return carry

        lax.fori_loop(0, NBPC, batch_body, 0)

    return k(xt)


def _tc_body(kern_ref, wproj_ref, wp_ref, wn_ref, bp_ref, bn_ref,
             coef_ref, out_ref):
    wproj = wproj_ref[...]
    A = wproj[:, :DIM]
    Bm = wproj[:, DIM:]
    kern = kern_ref[...]
    K2 = lax.dot_general(kern, A, (((1,), (1,)), ((), ())),
                         preferred_element_type=jnp.float32)
    up = jnp.sum(Bm * wp_ref[...], axis=1)
    un = jnp.sum(Bm * wn_ref[...], axis=1)
    cp = jnp.sum(Bm * bp_ref[...], axis=1)
    cn = jnp.sum(Bm * bn_ref[...], axis=1)
    M = jnp.concatenate(
        [K2, un[None, :], (up - un)[None, :], (cp - cn)[None, :],
         cn[None, :], jnp.zeros((3, DIM), jnp.float32)], axis=0)
    out_ref[0] = lax.dot_general(
        coef_ref[0], M, (((1,), (0,)), ((), ())),
        preferred_element_type=jnp.float32)


def _tc_combine(coef, kern, Wproj, wp_row, wn_row, bp_row, bn_row):
    def wspec(shape):
        return pl.BlockSpec(shape, lambda b, n: (0,) * len(shape))
    return pl.pallas_call(
        _tc_body,
        grid=(B, N // BLK),
        in_specs=[
            wspec((9, DIM)),
            wspec((DIM, 2 * DIM)),
            wspec((1, DIM)),
            wspec((1, DIM)),
            wspec((1, DIM)),
            wspec((1, DIM)),
            pl.BlockSpec((1, BLK, 16), lambda b, n: (b, n, 0)),
        ],
        out_specs=pl.BlockSpec((1, BLK, DIM), lambda b, n: (b, n, 0)),
        out_shape=jax.ShapeDtypeStruct((B, N, DIM), jnp.float32),
    )(kern, Wproj, wp_row, wn_row, bp_row, bn_row, coef)


def kernel(x, kernel, Wp, bp, Wn, bn, Wproj):
    xt = jnp.transpose(x, (2, 0, 1))
    coef = _sc_coef(xt).reshape(B, N, 16)
    return _tc_combine(coef, kernel, Wproj,
                       Wp[:, 0].reshape(1, DIM), Wn[:, 0].reshape(1, DIM),
                       bp.reshape(1, DIM), bn.reshape(1, DIM))


# speedup vs baseline: 8.4224x; 3.6310x over previous
import functools

import jax
import jax.numpy as jnp
from jax import lax
from jax.experimental import pallas as pl
from jax.experimental.pallas import tpu as pltpu
from jax.experimental.pallas import tpu_sc as plsc

H, W, DIM = 480, 640, 32
B, N = 16, 65536
PIX = H * W
NC, NS, L = 2, 16, 16
NBPC = B // NC
CHUNK = N // NS
NROW = CHUNK // 128
GS = PIX // NS
ZCH = 2400
TAPS = [(dy, dx) for dy in (-1, 0, 1) for dx in (-1, 0, 1)]
BLKN = 8192


def _sc_coef(xf):
    mesh = plsc.VectorSubcoreMesh(
        core_axis_name="c", subcore_axis_name="s",
        num_cores=NC, num_subcores=NS)

    @functools.partial(
        pl.kernel,
        out_type=jax.ShapeDtypeStruct((B, 16, N), jnp.float32),
        mesh=mesh,
        compiler_params=pltpu.CompilerParams(needs_layout_passes=False),
        scratch_types=[
            pltpu.VMEM((CHUNK * 4,), jnp.float32),
            pltpu.VMEM((CHUNK,), jnp.int32),
            pltpu.VMEM((CHUNK,), jnp.int32),
            pltpu.VMEM((NROW, 128), jnp.int32),
            pltpu.VMEM((NROW, 128), jnp.int32),
            pltpu.VMEM((16, CHUNK), jnp.float32),
            pltpu.VMEM((128,), jnp.float32),
            pltpu.VMEM((ZCH,), jnp.float32),
            pltpu.VMEM_SHARED((PIX,), jnp.float32),
            pltpu.SemaphoreType.DMA,
            pltpu.SemaphoreType.DMA,
        ],
    )
    def k(xf_hbm, out_hbm, xr_v, px_v, py_v, idx_v,
          nidx_v, coef_v, ones_v, zeros_v, grid_sh, sem_s, sem_g):
        c = lax.axis_index("c")
        s = lax.axis_index("s")
        base = s * CHUNK

        def fill_ones(i, carry):
            ones_v[pl.ds(i * L, L)] = jnp.full((L,), 1.0, jnp.float32)
            return carry
        lax.fori_loop(0, 128 // L, fill_ones, 0)

        def fill_zeros(i, carry):
            zeros_v[pl.ds(i * L, L)] = jnp.zeros((L,), jnp.float32)
            return carry
        lax.fori_loop(0, ZCH // L, fill_zeros, 0)

        def fill_const_rows(i, carry):
            sl = pl.ds(i * L, L)
            coef_v[12, sl] = jnp.full((L,), 1.0, jnp.float32)
            for row in (13, 14, 15):
                coef_v[row, sl] = jnp.zeros((L,), jnp.float32)
            return carry
        lax.fori_loop(0, CHUNK // L, fill_const_rows, 0)

        def batch_body(kk, carry):
            b = c * NBPC + kk
            def zloop(j, cz):
                pltpu.sync_copy(zeros_v,
                                grid_sh.at[pl.ds(s * GS + j * ZCH, ZCH)])
                return cz
            lax.fori_loop(0, GS // ZCH, zloop, 0)

            pltpu.sync_copy(xf_hbm.at[b, pl.ds(base * 4, CHUNK * 4)], xr_v)

            iota4 = lax.iota(jnp.int32, L) * 4
            def comp(i, cc):
                for k8 in range(8):
                    g = i * 8 + k8
                    sl = pl.ds(i * 128 + k8 * L, L)
                    fidx = g * (L * 4) + iota4
                    x0 = plsc.load_gather(xr_v, [fidx])
                    x1 = plsc.load_gather(xr_v, [fidx + 1])
                    t = plsc.load_gather(xr_v, [fidx + 2])
                    p = plsc.load_gather(xr_v, [fidx + 3])
                    px = jnp.clip((x0 * float(W - 1))
                                  .astype(jnp.int32), 0, W - 1)
                    py = jnp.clip((x1 * float(H - 1))
                                  .astype(jnp.int32), 0, H - 1)
                    px_v[sl] = px
                    py_v[sl] = py
                    idx_v[i, pl.ds(k8 * L, L)] = py * W + px
                    coef_v[9, sl] = t
                    coef_v[10, sl] = p * t
                    coef_v[11, sl] = p
                return cc
            lax.fori_loop(0, NROW, comp, 0)

            plsc.subcore_barrier()

            descs = [
                pltpu.async_copy(ones_v, grid_sh.at[idx_v.at[j]], sem_s,
                                 add=True)
                for j in range(NROW)
            ]
            for d in descs:
                d.wait()

            plsc.subcore_barrier()

            for tap, (dy, dx) in enumerate(TAPS):
                def nloop(i, cc, dy=dy, dx=dx):
                    for k8 in range(8):
                        sl = pl.ds(i * 128 + k8 * L, L)
                        px = px_v[sl]
                        py = py_v[sl]
                        if dy < 0:
                            ny = jnp.maximum(py - 1, 0)
                        elif dy > 0:
                            ny = jnp.minimum(py + 1, H - 1)
                        else:
                            ny = py
                        if dx < 0:
                            nx = jnp.maximum(px - 1, 0)
                        elif dx > 0:
                            nx = jnp.minimum(px + 1, W - 1)
                        else:
                            nx = px
                        nidx_v[i, pl.ds(k8 * L, L)] = ny * W + nx
                    return cc
                lax.fori_loop(0, NROW, nloop, 0)

                gds = [
                    pltpu.async_copy(grid_sh.at[nidx_v.at[j]],
                                     coef_v.at[tap, pl.ds(j * 128, 128)],
                                     sem_g)
                    for j in range(NROW)
                ]
                for d in gds:
                    d.wait()

            pltpu.sync_copy(
                coef_v, out_hbm.at[b, pl.ds(0, 16), pl.ds(base, CHUNK)])
            plsc.subcore_barrier()
            return carry

        lax.fori_loop(0, NBPC, batch_body, 0)

    return k(xf)


def _m_body(kern_ref, wproj_ref, wp_ref, wn_ref, bp_ref, bn_ref, mt_ref):
    wproj = wproj_ref[...]
    A = wproj[:, :DIM]
    Bm = wproj[:, DIM:]
    kern = kern_ref[...]
    K2T = lax.dot_general(A, kern, (((1,), (1,)), ((), ())),
                          preferred_element_type=jnp.float32)
    up = jnp.sum(Bm * wp_ref[...], axis=1)
    un = jnp.sum(Bm * wn_ref[...], axis=1)
    cp = jnp.sum(Bm * bp_ref[...], axis=1)
    cn = jnp.sum(Bm * bn_ref[...], axis=1)
    mt_ref[...] = jnp.concatenate(
        [K2T, un[:, None], (up - un)[:, None], (cp - cn)[:, None],
         cn[:, None], jnp.zeros((DIM, 3), jnp.float32)], axis=1)


def _tc_body(mt_ref, coef_ref, out_ref):
    out_ref[0] = lax.dot_general(
        mt_ref[...], coef_ref[0], (((1,), (0,)), ((), ())),
        preferred_element_type=jnp.float32)


def _tc_combine(coef, kern, Wproj, wp_row, wn_row, bp_row, bn_row):
    mt = pl.pallas_call(
        _m_body,
        out_shape=jax.ShapeDtypeStruct((DIM, 16), jnp.float32),
    )(kern, Wproj, wp_row, wn_row, bp_row, bn_row)
    return pl.pallas_call(
        _tc_body,
        grid=(B, N // BLKN),
        in_specs=[
            pl.BlockSpec((DIM, 16), lambda b, n: (0, 0)),
            pl.BlockSpec((1, 16, BLKN), lambda b, n: (b, 0, n)),
        ],
        out_specs=pl.BlockSpec((1, DIM, BLKN), lambda b, n: (b, 0, n)),
        out_shape=jax.ShapeDtypeStruct((B, DIM, N), jnp.float32),
    )(mt, coef)


def kernel(x, kernel, Wp, bp, Wn, bn, Wproj):
    xf = x.reshape(B, N * 4)
    coef = _sc_coef(xf)
    out_t = _tc_combine(coef, kernel, Wproj,
                        Wp[:, 0].reshape(1, DIM), Wn[:, 0].reshape(1, DIM),
                        bp.reshape(1, DIM), bn.reshape(1, DIM))
    return jnp.transpose(out_t, (0, 2, 1))


# speedup vs baseline: 9.8803x; 1.1731x over previous
import functools

import jax
import jax.numpy as jnp
from jax import lax
from jax.experimental import pallas as pl
from jax.experimental.pallas import tpu as pltpu
from jax.experimental.pallas import tpu_sc as plsc

H, W, DIM = 480, 640, 32
B, N = 16, 65536
PIX = H * W
NC, NS, L = 2, 16, 16
NBPC = B // NC
CHUNK = N // NS
NROW = CHUNK // 128
GS = PIX // NS
ZCH = 800
TAPS = [(dy, dx) for dy in (-1, 0, 1) for dx in (-1, 0, 1)]
BLKN = 8192


def _sc_coef(xp):
    mesh = plsc.VectorSubcoreMesh(
        core_axis_name="c", subcore_axis_name="s",
        num_cores=NC, num_subcores=NS)

    @functools.partial(
        pl.kernel,
        out_type=jax.ShapeDtypeStruct((B, 16, N), jnp.float32),
        mesh=mesh,
        compiler_params=pltpu.CompilerParams(needs_layout_passes=False),
        scratch_types=[
            pltpu.VMEM((CHUNK,), jnp.float32),
            pltpu.VMEM((CHUNK,), jnp.float32),
            pltpu.VMEM((9 * NROW, 128), jnp.int32),
            pltpu.VMEM((16, CHUNK), jnp.float32),
            pltpu.VMEM((128,), jnp.float32),
            pltpu.VMEM((ZCH,), jnp.float32),
            pltpu.VMEM_SHARED((PIX,), jnp.float32),
            pltpu.SemaphoreType.DMA,
            pltpu.SemaphoreType.DMA,
        ],
    )
    def k(xp_hbm, out_hbm, x0_v, x1_v, nidx_v, coef_v,
          ones_v, zeros_v, grid_sh, sem_s, sem_g):
        c = lax.axis_index("c")
        s = lax.axis_index("s")
        base = s * CHUNK

        def fill_ones(i, carry):
            ones_v[pl.ds(i * L, L)] = jnp.full((L,), 1.0, jnp.float32)
            return carry
        lax.fori_loop(0, 128 // L, fill_ones, 0)

        def fill_zeros(i, carry):
            zeros_v[pl.ds(i * L, L)] = jnp.zeros((L,), jnp.float32)
            return carry
        lax.fori_loop(0, ZCH // L, fill_zeros, 0)

        def fill_const_rows(i, carry):
            sl = pl.ds(i * L, L)
            coef_v[12, sl] = jnp.full((L,), 1.0, jnp.float32)
            for row in (13, 14, 15):
                coef_v[row, sl] = jnp.zeros((L,), jnp.float32)
            return carry
        lax.fori_loop(0, CHUNK // L, fill_const_rows, 0)

        def batch_body(kk, carry):
            b = c * NBPC + kk
            def zloop(j, cz):
                pltpu.sync_copy(zeros_v,
                                grid_sh.at[pl.ds(s * GS + j * ZCH, ZCH)])
                return cz
            lax.fori_loop(0, GS // ZCH, zloop, 0)

            pltpu.sync_copy(xp_hbm.at[b, pl.ds(0 * N + base, CHUNK)], x0_v)
            pltpu.sync_copy(xp_hbm.at[b, pl.ds(1 * N + base, CHUNK)], x1_v)
            pltpu.sync_copy(xp_hbm.at[b, pl.ds(2 * N + base, CHUNK)],
                            coef_v.at[9])
            pltpu.sync_copy(xp_hbm.at[b, pl.ds(3 * N + base, CHUNK)],
                            coef_v.at[11])

            def comp(i, cc):
                for k8 in range(8):
                    sl = pl.ds(i * 128 + k8 * L, L)
                    col = pl.ds(k8 * L, L)
                    px = jnp.clip((x0_v[sl] * float(W - 1))
                                  .astype(jnp.int32), 0, W - 1)
                    py = jnp.clip((x1_v[sl] * float(H - 1))
                                  .astype(jnp.int32), 0, H - 1)
                    yw = (jnp.maximum(py - 1, 0) * W,
                          py * W,
                          jnp.minimum(py + 1, H - 1) * W)
                    xs = (jnp.maximum(px - 1, 0),
                          px,
                          jnp.minimum(px + 1, W - 1))
                    for tap in range(9):
                        nidx_v[tap * NROW + i, col] = (yw[tap // 3]
                                                       + xs[tap % 3])
                    coef_v[10, sl] = coef_v[9, sl] * coef_v[11, sl]
                return cc
            lax.fori_loop(0, NROW, comp, 0)

            plsc.subcore_barrier()

            descs = [
                pltpu.async_copy(ones_v, grid_sh.at[nidx_v.at[4 * NROW + j]],
                                 sem_s, add=True)
                for j in range(NROW)
            ]
            for d in descs:
                d.wait()

            plsc.subcore_barrier()

            gds = [
                pltpu.async_copy(grid_sh.at[nidx_v.at[tap * NROW + j]],
                                 coef_v.at[tap, pl.ds(j * 128, 128)],
                                 sem_g)
                for tap in range(9)
                for j in range(NROW)
            ]
            for d in gds:
                d.wait()

            pltpu.sync_copy(
                coef_v, out_hbm.at[b, pl.ds(0, 16), pl.ds(base, CHUNK)])
            plsc.subcore_barrier()
            return carry

        lax.fori_loop(0, NBPC, batch_body, 0)

    return k(xp)


def _m_body(kern_ref, wproj_ref, wp_ref, wn_ref, bp_ref, bn_ref, mt_ref):
    wproj = wproj_ref[...]
    A = wproj[:, :DIM]
    Bm = wproj[:, DIM:]
    kern = kern_ref[...]
    K2T = lax.dot_general(A, kern, (((1,), (1,)), ((), ())),
                          preferred_element_type=jnp.float32)
    up = jnp.sum(Bm * wp_ref[...], axis=1)
    un = jnp.sum(Bm * wn_ref[...], axis=1)
    cp = jnp.sum(Bm * bp_ref[...], axis=1)
    cn = jnp.sum(Bm * bn_ref[...], axis=1)
    mt_ref[...] = jnp.concatenate(
        [K2T, un[:, None], (up - un)[:, None], (cp - cn)[:, None],
         cn[:, None], jnp.zeros((DIM, 3), jnp.float32)], axis=1)


def _tc_body(mt_ref, coef_ref, out_ref):
    out_ref[0] = lax.dot_general(
        mt_ref[...], coef_ref[0], (((1,), (0,)), ((), ())),
        preferred_element_type=jnp.float32)


def _tc_combine(coef, kern, Wproj, wp_row, wn_row, bp_row, bn_row):
    mt = pl.pallas_call(
        _m_body,
        out_shape=jax.ShapeDtypeStruct((DIM, 16), jnp.float32),
    )(kern, Wproj, wp_row, wn_row, bp_row, bn_row)
    return pl.pallas_call(
        _tc_body,
        grid=(B, N // BLKN),
        in_specs=[
            pl.BlockSpec((DIM, 16), lambda b, n: (0, 0)),
            pl.BlockSpec((1, 16, BLKN), lambda b, n: (b, 0, n)),
        ],
        out_specs=pl.BlockSpec((1, DIM, BLKN), lambda b, n: (b, 0, n)),
        out_shape=jax.ShapeDtypeStruct((B, DIM, N), jnp.float32),
    )(mt, coef)


def kernel(x, kernel, Wp, bp, Wn, bn, Wproj):
    xp = jnp.transpose(x, (0, 2, 1)).reshape(B, 4 * N)
    coef = _sc_coef(xp)
    out_t = _tc_combine(coef, kernel, Wproj,
                        Wp[:, 0].reshape(1, DIM), Wn[:, 0].reshape(1, DIM),
                        bp.reshape(1, DIM), bn.reshape(1, DIM))
    return jnp.transpose(out_t, (0, 2, 1))


# speedup vs baseline: 10.0453x; 1.0167x over previous
import functools

import jax
import jax.numpy as jnp
from jax import lax
from jax.experimental import pallas as pl
from jax.experimental.pallas import tpu as pltpu
from jax.experimental.pallas import tpu_sc as plsc

H, W, DIM = 480, 640, 32
B, N = 16, 65536
PIX = H * W
NC, NS, L = 2, 16, 16
NBPC = B // NC
CHUNK = N // NS
NROW = CHUNK // 128
GS = PIX // NS
ZCH = 800
TAPS = [(dy, dx) for dy in (-1, 0, 1) for dx in (-1, 0, 1)]
BLKN = 8192


def _sc_coef(xp):
    mesh = plsc.VectorSubcoreMesh(
        core_axis_name="c", subcore_axis_name="s",
        num_cores=NC, num_subcores=NS)

    @functools.partial(
        pl.kernel,
        out_type=jax.ShapeDtypeStruct((B, 16, N), jnp.float32),
        mesh=mesh,
        compiler_params=pltpu.CompilerParams(needs_layout_passes=False),
        scratch_types=[
            pltpu.VMEM((CHUNK,), jnp.float32),
            pltpu.VMEM((CHUNK,), jnp.float32),
            pltpu.VMEM((9 * NROW, 128), jnp.int32),
            pltpu.VMEM((16, CHUNK), jnp.float32),
            pltpu.VMEM((128,), jnp.float32),
            pltpu.VMEM((128,), jnp.float32),
            pltpu.VMEM((ZCH,), jnp.float32),
            pltpu.VMEM_SHARED((PIX,), jnp.float32),
            pltpu.SemaphoreType.DMA,
            pltpu.SemaphoreType.DMA,
        ],
    )
    def k(xp_hbm, out_hbm, x0_v, x1_v, nidx_v, coef_v,
          ones_v, mones_v, zeros_v, grid_sh, sem_s, sem_g):
        c = lax.axis_index("c")
        s = lax.axis_index("s")
        base = s * CHUNK

        def fill_ones(i, carry):
            ones_v[pl.ds(i * L, L)] = jnp.full((L,), 1.0, jnp.float32)
            mones_v[pl.ds(i * L, L)] = jnp.full((L,), -1.0, jnp.float32)
            return carry
        lax.fori_loop(0, 128 // L, fill_ones, 0)

        def fill_zeros(i, carry):
            zeros_v[pl.ds(i * L, L)] = jnp.zeros((L,), jnp.float32)
            return carry
        lax.fori_loop(0, ZCH // L, fill_zeros, 0)

        def fill_const_rows(i, carry):
            sl = pl.ds(i * L, L)
            coef_v[12, sl] = jnp.full((L,), 1.0, jnp.float32)
            for row in (13, 14, 15):
                coef_v[row, sl] = jnp.zeros((L,), jnp.float32)
            return carry
        lax.fori_loop(0, CHUNK // L, fill_const_rows, 0)

        def zloop(j, cz):
            pltpu.sync_copy(zeros_v,
                            grid_sh.at[pl.ds(s * GS + j * ZCH, ZCH)])
            return cz
        lax.fori_loop(0, GS // ZCH, zloop, 0)

        def batch_body(kk, carry):
            b = c * NBPC + kk
            pltpu.sync_copy(xp_hbm.at[b, pl.ds(0 * N + base, CHUNK)], x0_v)
            pltpu.sync_copy(xp_hbm.at[b, pl.ds(1 * N + base, CHUNK)], x1_v)
            pltpu.sync_copy(xp_hbm.at[b, pl.ds(2 * N + base, CHUNK)],
                            coef_v.at[9])
            pltpu.sync_copy(xp_hbm.at[b, pl.ds(3 * N + base, CHUNK)],
                            coef_v.at[11])

            def comp(i, cc):
                for k8 in range(8):
                    sl = pl.ds(i * 128 + k8 * L, L)
                    col = pl.ds(k8 * L, L)
                    px = jnp.clip((x0_v[sl] * float(W - 1))
                                  .astype(jnp.int32), 0, W - 1)
                    py = jnp.clip((x1_v[sl] * float(H - 1))
                                  .astype(jnp.int32), 0, H - 1)
                    yw = (jnp.maximum(py - 1, 0) * W,
                          py * W,
                          jnp.minimum(py + 1, H - 1) * W)
                    xs = (jnp.maximum(px - 1, 0),
                          px,
                          jnp.minimum(px + 1, W - 1))
                    for tap in range(9):
                        nidx_v[tap * NROW + i, col] = (yw[tap // 3]
                                                       + xs[tap % 3])
                    coef_v[10, sl] = coef_v[9, sl] * coef_v[11, sl]
                return cc
            lax.fori_loop(0, NROW, comp, 0)

            plsc.subcore_barrier()

            descs = [
                pltpu.async_copy(ones_v, grid_sh.at[nidx_v.at[4 * NROW + j]],
                                 sem_s, add=True)
                for j in range(NROW)
            ]
            for d in descs:
                d.wait()

            plsc.subcore_barrier()

            gds = [
                pltpu.async_copy(grid_sh.at[nidx_v.at[tap * NROW + j]],
                                 coef_v.at[tap, pl.ds(j * 128, 128)],
                                 sem_g)
                for tap in range(9)
                for j in range(NROW)
            ]
            for d in gds:
                d.wait()

            plsc.subcore_barrier()

            uds = [
                pltpu.async_copy(mones_v, grid_sh.at[nidx_v.at[4 * NROW + j]],
                                 sem_s, add=True)
                for j in range(NROW)
            ]
            pltpu.sync_copy(
                coef_v, out_hbm.at[b, pl.ds(0, 16), pl.ds(base, CHUNK)])
            for d in uds:
                d.wait()
            plsc.subcore_barrier()
            return carry

        lax.fori_loop(0, NBPC, batch_body, 0)

    return k(xp)


def _m_body(kern_ref, wproj_ref, wp_ref, wn_ref, bp_ref, bn_ref, mt_ref):
    wproj = wproj_ref[...]
    A = wproj[:, :DIM]
    Bm = wproj[:, DIM:]
    kern = kern_ref[...]
    K2T = lax.dot_general(A, kern, (((1,), (1,)), ((), ())),
                          preferred_element_type=jnp.float32)
    up = jnp.sum(Bm * wp_ref[...], axis=1)
    un = jnp.sum(Bm * wn_ref[...], axis=1)
    cp = jnp.sum(Bm * bp_ref[...], axis=1)
    cn = jnp.sum(Bm * bn_ref[...], axis=1)
    mt_ref[...] = jnp.concatenate(
        [K2T, un[:, None], (up - un)[:, None], (cp - cn)[:, None],
         cn[:, None], jnp.zeros((DIM, 3), jnp.float32)], axis=1)


def _tc_body(mt_ref, coef_ref, out_ref):
    out_ref[0] = lax.dot_general(
        mt_ref[...], coef_ref[0], (((1,), (0,)), ((), ())),
        preferred_element_type=jnp.float32)


def _tc_combine(coef, kern, Wproj, wp_row, wn_row, bp_row, bn_row):
    mt = pl.pallas_call(
        _m_body,
        out_shape=jax.ShapeDtypeStruct((DIM, 16), jnp.float32),
    )(kern, Wproj, wp_row, wn_row, bp_row, bn_row)
    return pl.pallas_call(
        _tc_body,
        grid=(B, N // BLKN),
        in_specs=[
            pl.BlockSpec((DIM, 16), lambda b, n: (0, 0)),
            pl.BlockSpec((1, 16, BLKN), lambda b, n: (b, 0, n)),
        ],
        out_specs=pl.BlockSpec((1, DIM, BLKN), lambda b, n: (b, 0, n)),
        out_shape=jax.ShapeDtypeStruct((B, DIM, N), jnp.float32),
    )(mt, coef)


def kernel(x, kernel, Wp, bp, Wn, bn, Wproj):
    xp = jnp.transpose(x, (0, 2, 1)).reshape(B, 4 * N)
    coef = _sc_coef(xp)
    out_t = _tc_combine(coef, kernel, Wproj,
                        Wp[:, 0].reshape(1, DIM), Wn[:, 0].reshape(1, DIM),
                        bp.reshape(1, DIM), bn.reshape(1, DIM))
    return jnp.transpose(out_t, (0, 2, 1))


# speedup vs baseline: 11.3568x; 1.1306x over previous
import functools

import jax
import jax.numpy as jnp
from jax import lax
from jax.experimental import pallas as pl
from jax.experimental.pallas import tpu as pltpu
from jax.experimental.pallas import tpu_sc as plsc

H, W, DIM = 480, 640, 32
B, N = 16, 65536
PIX = H * W
NC, NS, L = 2, 16, 16
NBPC = B // NC
CHUNK = N // NS
NROW = CHUNK // 128
GS = PIX // NS
ZCH = 800
TAPS = [(dy, dx) for dy in (-1, 0, 1) for dx in (-1, 0, 1)]
BLKN = 8192


NBG = 2
GB = B // NBG
NBPG = GB // NC


def _sc_coef(xp, g):
    mesh = plsc.VectorSubcoreMesh(
        core_axis_name="c", subcore_axis_name="s",
        num_cores=NC, num_subcores=NS)

    @functools.partial(
        pl.kernel,
        out_type=jax.ShapeDtypeStruct((GB, 16, N), jnp.float32),
        mesh=mesh,
        compiler_params=pltpu.CompilerParams(needs_layout_passes=False),
        scratch_types=[
            pltpu.VMEM((CHUNK,), jnp.float32),
            pltpu.VMEM((CHUNK,), jnp.float32),
            pltpu.VMEM((9 * NROW, 128), jnp.int32),
            pltpu.VMEM((16, CHUNK), jnp.float32),
            pltpu.VMEM((128,), jnp.float32),
            pltpu.VMEM((128,), jnp.float32),
            pltpu.VMEM((ZCH,), jnp.float32),
            pltpu.VMEM_SHARED((PIX,), jnp.float32),
            pltpu.SemaphoreType.DMA,
            pltpu.SemaphoreType.DMA,
        ],
    )
    def k(xp_hbm, out_hbm, x0_v, x1_v, nidx_v, coef_v,
          ones_v, mones_v, zeros_v, grid_sh, sem_s, sem_g):
        c = lax.axis_index("c")
        s = lax.axis_index("s")
        base = s * CHUNK

        def fill_ones(i, carry):
            ones_v[pl.ds(i * L, L)] = jnp.full((L,), 1.0, jnp.float32)
            mones_v[pl.ds(i * L, L)] = jnp.full((L,), -1.0, jnp.float32)
            return carry
        lax.fori_loop(0, 128 // L, fill_ones, 0)

        def fill_zeros(i, carry):
            zeros_v[pl.ds(i * L, L)] = jnp.zeros((L,), jnp.float32)
            return carry
        lax.fori_loop(0, ZCH // L, fill_zeros, 0)

        def fill_const_rows(i, carry):
            sl = pl.ds(i * L, L)
            coef_v[12, sl] = jnp.full((L,), 1.0, jnp.float32)
            for row in (13, 14, 15):
                coef_v[row, sl] = jnp.zeros((L,), jnp.float32)
            return carry
        lax.fori_loop(0, CHUNK // L, fill_const_rows, 0)

        def zloop(j, cz):
            pltpu.sync_copy(zeros_v,
                            grid_sh.at[pl.ds(s * GS + j * ZCH, ZCH)])
            return cz
        lax.fori_loop(0, GS // ZCH, zloop, 0)

        def batch_body(kk, carry):
            lb = kk * NC + c
            b = g * GB + lb
            pltpu.sync_copy(xp_hbm.at[b, pl.ds(0 * N + base, CHUNK)], x0_v)
            pltpu.sync_copy(xp_hbm.at[b, pl.ds(1 * N + base, CHUNK)], x1_v)
            pltpu.sync_copy(xp_hbm.at[b, pl.ds(2 * N + base, CHUNK)],
                            coef_v.at[9])
            pltpu.sync_copy(xp_hbm.at[b, pl.ds(3 * N + base, CHUNK)],
                            coef_v.at[11])

            def comp(i, cc):
                for k8 in range(8):
                    sl = pl.ds(i * 128 + k8 * L, L)
                    col = pl.ds(k8 * L, L)
                    px = jnp.clip((x0_v[sl] * float(W - 1))
                                  .astype(jnp.int32), 0, W - 1)
                    py = jnp.clip((x1_v[sl] * float(H - 1))
                                  .astype(jnp.int32), 0, H - 1)
                    yw = (jnp.maximum(py - 1, 0) * W,
                          py * W,
                          jnp.minimum(py + 1, H - 1) * W)
                    xs = (jnp.maximum(px - 1, 0),
                          px,
                          jnp.minimum(px + 1, W - 1))
                    for tap in range(9):
                        nidx_v[tap * NROW + i, col] = (yw[tap // 3]
                                                       + xs[tap % 3])
                    coef_v[10, sl] = coef_v[9, sl] * coef_v[11, sl]
                return cc
            lax.fori_loop(0, NROW, comp, 0)

            plsc.subcore_barrier()

            descs = [
                pltpu.async_copy(ones_v, grid_sh.at[nidx_v.at[4 * NROW + j]],
                                 sem_s, add=True)
                for j in range(NROW)
            ]
            for d in descs:
                d.wait()

            plsc.subcore_barrier()

            gds = [
                pltpu.async_copy(grid_sh.at[nidx_v.at[tap * NROW + j]],
                                 coef_v.at[tap, pl.ds(j * 128, 128)],
                                 sem_g)
                for tap in range(9)
                for j in range(NROW)
            ]
            for d in gds:
                d.wait()

            plsc.subcore_barrier()

            uds = [
                pltpu.async_copy(mones_v, grid_sh.at[nidx_v.at[4 * NROW + j]],
                                 sem_s, add=True)
                for j in range(NROW)
            ]
            pltpu.sync_copy(
                coef_v, out_hbm.at[lb, pl.ds(0, 16), pl.ds(base, CHUNK)])
            for d in uds:
                d.wait()
            plsc.subcore_barrier()
            return carry

        lax.fori_loop(0, NBPG, batch_body, 0)

    return k(xp)


def _m_body(kern_ref, wproj_ref, wp_ref, wn_ref, bp_ref, bn_ref, mt_ref):
    wproj = wproj_ref[...]
    A = wproj[:, :DIM]
    Bm = wproj[:, DIM:]
    kern = kern_ref[...]
    K2T = lax.dot_general(A, kern, (((1,), (1,)), ((), ())),
                          preferred_element_type=jnp.float32)
    up = jnp.sum(Bm * wp_ref[...], axis=1)
    un = jnp.sum(Bm * wn_ref[...], axis=1)
    cp = jnp.sum(Bm * bp_ref[...], axis=1)
    cn = jnp.sum(Bm * bn_ref[...], axis=1)
    mt_ref[...] = jnp.concatenate(
        [K2T, un[:, None], (up - un)[:, None], (cp - cn)[:, None],
         cn[:, None], jnp.zeros((DIM, 3), jnp.float32)], axis=1)


def _tc_body(mt_ref, coef_ref, out_ref):
    out_ref[0] = lax.dot_general(
        mt_ref[...], coef_ref[0], (((1,), (0,)), ((), ())),
        preferred_element_type=jnp.float32)


def _tc_body_acc(mt_ref, coef_ref, prev_ref, out_ref):
    out_ref[0] = lax.dot_general(
        mt_ref[...], coef_ref[0], (((1,), (0,)), ((), ())),
        preferred_element_type=jnp.float32)


def _tc_combine(coefs, kern, Wproj, wp_row, wn_row, bp_row, bn_row):
    mt = pl.pallas_call(
        _m_body,
        out_shape=jax.ShapeDtypeStruct((DIM, 16), jnp.float32),
    )(kern, Wproj, wp_row, wn_row, bp_row, bn_row)
    out = pl.pallas_call(
        _tc_body,
        grid=(GB, N // BLKN),
        in_specs=[
            pl.BlockSpec((DIM, 16), lambda b, n: (0, 0)),
            pl.BlockSpec((1, 16, BLKN), lambda b, n: (b, 0, n)),
        ],
        out_specs=pl.BlockSpec((1, DIM, BLKN), lambda b, n: (b, 0, n)),
        out_shape=jax.ShapeDtypeStruct((B, DIM, N), jnp.float32),
    )(mt, coefs[0])
    for g in range(1, NBG):
        out = pl.pallas_call(
            _tc_body_acc,
            grid=(GB, N // BLKN),
            in_specs=[
                pl.BlockSpec((DIM, 16), lambda b, n: (0, 0)),
                pl.BlockSpec((1, 16, BLKN), lambda b, n: (b, 0, n)),
                pl.BlockSpec(memory_space=pl.ANY),
            ],
            out_specs=pl.BlockSpec((1, DIM, BLKN),
                                   lambda b, n, g=g: (b + g * GB, 0, n)),
            out_shape=jax.ShapeDtypeStruct((B, DIM, N), jnp.float32),
            input_output_aliases={2: 0},
        )(mt, coefs[g], out)
    return out


def kernel(x, kernel, Wp, bp, Wn, bn, Wproj):
    xp = jnp.transpose(x, (0, 2, 1)).reshape(B, 4 * N)
    coefs = [_sc_coef(xp, g) for g in range(NBG)]
    out_t = _tc_combine(coefs, kernel, Wproj,
                        Wp[:, 0].reshape(1, DIM), Wn[:, 0].reshape(1, DIM),
                        bp.reshape(1, DIM), bn.reshape(1, DIM))
    return jnp.transpose(out_t, (0, 2, 1))


# speedup vs baseline: 11.5139x; 1.0138x over previous
import functools

import jax
import jax.numpy as jnp
from jax import lax
from jax.experimental import pallas as pl
from jax.experimental.pallas import tpu as pltpu
from jax.experimental.pallas import tpu_sc as plsc

H, W, DIM = 480, 640, 32
B, N = 16, 65536
PIX = H * W
NC, NS, L = 2, 16, 16
NBPC = B // NC
CHUNK = N // NS
NROW = CHUNK // 128
GS = PIX // NS
ZCH = 800
TAPS = [(dy, dx) for dy in (-1, 0, 1) for dx in (-1, 0, 1)]
BLKN = 8192


NBG = 4
GB = B // NBG
NBPG = GB // NC


def _sc_coef(xp, g):
    mesh = plsc.VectorSubcoreMesh(
        core_axis_name="c", subcore_axis_name="s",
        num_cores=NC, num_subcores=NS)

    @functools.partial(
        pl.kernel,
        out_type=jax.ShapeDtypeStruct((GB, 16, N), jnp.float32),
        mesh=mesh,
        compiler_params=pltpu.CompilerParams(needs_layout_passes=False),
        scratch_types=[
            pltpu.VMEM((CHUNK,), jnp.float32),
            pltpu.VMEM((CHUNK,), jnp.float32),
            pltpu.VMEM((9 * NROW, 128), jnp.int32),
            pltpu.VMEM((16, CHUNK), jnp.float32),
            pltpu.VMEM((128,), jnp.float32),
            pltpu.VMEM((128,), jnp.float32),
            pltpu.VMEM((ZCH,), jnp.float32),
            pltpu.VMEM_SHARED((PIX,), jnp.float32),
            pltpu.SemaphoreType.DMA,
            pltpu.SemaphoreType.DMA,
        ],
    )
    def k(xp_hbm, out_hbm, x0_v, x1_v, nidx_v, coef_v,
          ones_v, mones_v, zeros_v, grid_sh, sem_s, sem_g):
        c = lax.axis_index("c")
        s = lax.axis_index("s")
        base = s * CHUNK

        def fill_ones(i, carry):
            ones_v[pl.ds(i * L, L)] = jnp.full((L,), 1.0, jnp.float32)
            mones_v[pl.ds(i * L, L)] = jnp.full((L,), -1.0, jnp.float32)
            return carry
        lax.fori_loop(0, 128 // L, fill_ones, 0)

        def fill_zeros(i, carry):
            zeros_v[pl.ds(i * L, L)] = jnp.zeros((L,), jnp.float32)
            return carry
        lax.fori_loop(0, ZCH // L, fill_zeros, 0)

        def fill_const_rows(i, carry):
            sl = pl.ds(i * L, L)
            coef_v[12, sl] = jnp.full((L,), 1.0, jnp.float32)
            for row in (13, 14, 15):
                coef_v[row, sl] = jnp.zeros((L,), jnp.float32)
            return carry
        lax.fori_loop(0, CHUNK // L, fill_const_rows, 0)

        def zloop(j, cz):
            pltpu.sync_copy(zeros_v,
                            grid_sh.at[pl.ds(s * GS + j * ZCH, ZCH)])
            return cz
        lax.fori_loop(0, GS // ZCH, zloop, 0)

        def batch_body(kk, carry):
            lb = kk * NC + c
            b = g * GB + lb
            pltpu.sync_copy(xp_hbm.at[b, pl.ds(0 * N + base, CHUNK)], x0_v)
            pltpu.sync_copy(xp_hbm.at[b, pl.ds(1 * N + base, CHUNK)], x1_v)
            pltpu.sync_copy(xp_hbm.at[b, pl.ds(2 * N + base, CHUNK)],
                            coef_v.at[9])
            pltpu.sync_copy(xp_hbm.at[b, pl.ds(3 * N + base, CHUNK)],
                            coef_v.at[11])

            def comp(i, cc):
                for k8 in range(8):
                    sl = pl.ds(i * 128 + k8 * L, L)
                    col = pl.ds(k8 * L, L)
                    px = jnp.clip((x0_v[sl] * float(W - 1))
                                  .astype(jnp.int32), 0, W - 1)
                    py = jnp.clip((x1_v[sl] * float(H - 1))
                                  .astype(jnp.int32), 0, H - 1)
                    yw = (jnp.maximum(py - 1, 0) * W,
                          py * W,
                          jnp.minimum(py + 1, H - 1) * W)
                    xs = (jnp.maximum(px - 1, 0),
                          px,
                          jnp.minimum(px + 1, W - 1))
                    for tap in range(9):
                        nidx_v[tap * NROW + i, col] = (yw[tap // 3]
                                                       + xs[tap % 3])
                    coef_v[10, sl] = coef_v[9, sl] * coef_v[11, sl]
                return cc
            lax.fori_loop(0, NROW, comp, 0)

            plsc.subcore_barrier()

            descs = [
                pltpu.async_copy(ones_v, grid_sh.at[nidx_v.at[4 * NROW + j]],
                                 sem_s, add=True)
                for j in range(NROW)
            ]
            for d in descs:
                d.wait()

            plsc.subcore_barrier()

            gds = [
                pltpu.async_copy(grid_sh.at[nidx_v.at[tap * NROW + j]],
                                 coef_v.at[tap, pl.ds(j * 128, 128)],
                                 sem_g)
                for tap in range(9)
                for j in range(NROW)
            ]
            for d in gds:
                d.wait()

            plsc.subcore_barrier()

            uds = [
                pltpu.async_copy(mones_v, grid_sh.at[nidx_v.at[4 * NROW + j]],
                                 sem_s, add=True)
                for j in range(NROW)
            ]
            pltpu.sync_copy(
                coef_v, out_hbm.at[lb, pl.ds(0, 16), pl.ds(base, CHUNK)])
            for d in uds:
                d.wait()
            plsc.subcore_barrier()
            return carry

        lax.fori_loop(0, NBPG, batch_body, 0)

    return k(xp)


def _m_body(kern_ref, wproj_ref, wp_ref, wn_ref, bp_ref, bn_ref, mt_ref):
    wproj = wproj_ref[...]
    A = wproj[:, :DIM]
    Bm = wproj[:, DIM:]
    kern = kern_ref[...]
    K2T = lax.dot_general(A, kern, (((1,), (1,)), ((), ())),
                          preferred_element_type=jnp.float32)
    up = jnp.sum(Bm * wp_ref[...], axis=1)
    un = jnp.sum(Bm * wn_ref[...], axis=1)
    cp = jnp.sum(Bm * bp_ref[...], axis=1)
    cn = jnp.sum(Bm * bn_ref[...], axis=1)
    mt_ref[...] = jnp.concatenate(
        [K2T, un[:, None], (up - un)[:, None], (cp - cn)[:, None],
         cn[:, None], jnp.zeros((DIM, 3), jnp.float32)], axis=1)


def _tc_body(mt_ref, coef_ref, out_ref):
    out_ref[0] = lax.dot_general(
        mt_ref[...], coef_ref[0], (((1,), (0,)), ((), ())),
        preferred_element_type=jnp.float32)


def _tc_body_acc(mt_ref, coef_ref, prev_ref, out_ref):
    out_ref[0] = lax.dot_general(
        mt_ref[...], coef_ref[0], (((1,), (0,)), ((), ())),
        preferred_element_type=jnp.float32)


def _tc_combine(coefs, kern, Wproj, wp_row, wn_row, bp_row, bn_row):
    mt = pl.pallas_call(
        _m_body,
        out_shape=jax.ShapeDtypeStruct((DIM, 16), jnp.float32),
    )(kern, Wproj, wp_row, wn_row, bp_row, bn_row)
    out = pl.pallas_call(
        _tc_body,
        grid=(GB, N // BLKN),
        in_specs=[
            pl.BlockSpec((DIM, 16), lambda b, n: (0, 0)),
            pl.BlockSpec((1, 16, BLKN), lambda b, n: (b, 0, n)),
        ],
        out_specs=pl.BlockSpec((1, DIM, BLKN), lambda b, n: (b, 0, n)),
        out_shape=jax.ShapeDtypeStruct((B, DIM, N), jnp.float32),
    )(mt, coefs[0])
    for g in range(1, NBG):
        out = pl.pallas_call(
            _tc_body_acc,
            grid=(GB, N // BLKN),
            in_specs=[
                pl.BlockSpec((DIM, 16), lambda b, n: (0, 0)),
                pl.BlockSpec((1, 16, BLKN), lambda b, n: (b, 0, n)),
                pl.BlockSpec(memory_space=pl.ANY),
            ],
            out_specs=pl.BlockSpec((1, DIM, BLKN),
                                   lambda b, n, g=g: (b + g * GB, 0, n)),
            out_shape=jax.ShapeDtypeStruct((B, DIM, N), jnp.float32),
            input_output_aliases={2: 0},
        )(mt, coefs[g], out)
    return out


def kernel(x, kernel, Wp, bp, Wn, bn, Wproj):
    xp = jnp.transpose(x, (0, 2, 1)).reshape(B, 4 * N)
    coefs = [_sc_coef(xp, g) for g in range(NBG)]
    out_t = _tc_combine(coefs, kernel, Wproj,
                        Wp[:, 0].reshape(1, DIM), Wn[:, 0].reshape(1, DIM),
                        bp.reshape(1, DIM), bn.reshape(1, DIM))
    return jnp.transpose(out_t, (0, 2, 1))
